# Initial kernel scaffold; baseline (speedup 1.0000x reference)
import jax
import jax.numpy as jnp
from jax.experimental import pallas as pl


def kernel(charge, sigma, bond_dist, edge_index):
    raise NotImplementedError("write your pallas kernel here")


# speedup vs baseline: 154.7275x; 154.7275x over previous
import functools

import jax
import jax.numpy as jnp
from jax import lax
from jax.experimental import pallas as pl
from jax.experimental.pallas import tpu as pltpu
from jax.experimental.pallas import tpu_sc as plsc

COULOMB_K = 14.399645351950548
NUM_NODES = 100000
NUM_EDGES = 6400000

NC, NS = 2, 16
NW = NC * NS
ROW = 128
K = 16
CHUNK = K * ROW
NROWS = NUM_EDGES // ROW
NCHUNKS = NROWS // K

_INV_SQRT2 = 0.7071067811865476
_ERF_P = 0.3275911
_ERF_A1 = 0.254829592
_ERF_A2 = -0.284496736
_ERF_A3 = 1.421413741
_ERF_A4 = -1.453152027
_ERF_A5 = 1.061405429


def _edge_math(si, di, d, q, sigma_v):
  ss = plsc.load_gather(sigma_v, [si])
  sd = plsc.load_gather(sigma_v, [di])
  g2 = ss * ss + sd * sd
  ii = plsc.bitcast(g2, jnp.int32)
  ii = jnp.int32(0x5F3759DF) - lax.shift_right_logical(ii, 1)
  y = plsc.bitcast(ii, jnp.float32)
  for _ in range(3):
    y = y * (1.5 - 0.5 * g2 * y * y)
  x = d * _INV_SQRT2 * y
  t = 1.0 / (1.0 + _ERF_P * x)
  e2 = jnp.exp(-(x * x))
  p = t * (_ERF_A1 + t * (_ERF_A2 + t * (_ERF_A3 + t * (_ERF_A4 + t * _ERF_A5))))
  erf_x = 1.0 - p * e2
  pc = 1.0 + d * d * d * (-10.0 + d * (15.0 - 6.0 * d))
  return (COULOMB_K * q) * erf_x * pc / d


def _body(src_hbm, dst_hbm, bond_hbm, charge_hbm, sigma_hbm, zeros_hbm,
          out_hbm, sigma_v, srcb, dstb, bondb, qb, potb, vshared, gsem):
  c = lax.axis_index("c")
  s = lax.axis_index("s")
  wid = s * NC + c

  pltpu.sync_copy(sigma_hbm, sigma_v)

  @pl.when(s == 0)
  def _():
    pltpu.sync_copy(zeros_hbm, vshared)

  plsc.subcore_barrier()

  c0 = wid * NCHUNKS // NW
  c1 = (wid + 1) * NCHUNKS // NW

  def chunk_body(i, carry):
    b = i * K
    pltpu.sync_copy(src_hbm.at[pl.ds(b, K)], srcb)
    pltpu.sync_copy(dst_hbm.at[pl.ds(b, K)], dstb)
    pltpu.sync_copy(bond_hbm.at[pl.ds(b, K)], bondb)

    descs = []
    for r in range(K):
      dsc = pltpu.make_async_copy(charge_hbm.at[dstb.at[r]], qb.at[r], gsem)
      dsc.start()
      descs.append(dsc)
    for dsc in descs:
      dsc.wait()

    def row_body(r, carry2):
      for j in range(ROW // 16):
        sl = pl.ds(j * 16, 16)
        si = srcb[r, sl]
        di = dstb[r, sl]
        d = bondb[r, sl]
        q = qb[r, sl]
        potb[r, sl] = _edge_math(si, di, d, q, sigma_v)
      return carry2

    lax.fori_loop(0, K, row_body, 0)

    sdescs = []
    for r in range(K):
      dsc = pltpu.make_async_copy(potb.at[r], vshared.at[dstb.at[r]], gsem)
      dsc.start(add=True)
      sdescs.append(dsc)
    for dsc in sdescs:
      dsc.wait()
    return carry

  lax.fori_loop(c0, c1, chunk_body, 0)

  plsc.subcore_barrier()

  @pl.when(s == 0)
  def _():
    pltpu.sync_copy(vshared, out_hbm.at[c])


@jax.jit
def kernel(charge, sigma, bond_dist, edge_index):
  src2d = edge_index[0].reshape(NROWS, ROW)
  dst2d = edge_index[1].reshape(NROWS, ROW)
  bond2d = bond_dist.reshape(NROWS, ROW)
  zeros = jnp.zeros((NUM_NODES,), jnp.float32)

  mesh = plsc.VectorSubcoreMesh(core_axis_name="c", subcore_axis_name="s")
  f = pl.kernel(
      _body,
      out_type=jax.ShapeDtypeStruct((NC, NUM_NODES), jnp.float32),
      mesh=mesh,
      compiler_params=pltpu.CompilerParams(needs_layout_passes=False),
      scratch_types=[
          pltpu.VMEM((NUM_NODES,), jnp.float32),
          pltpu.VMEM((K, ROW), jnp.int32),
          pltpu.VMEM((K, ROW), jnp.int32),
          pltpu.VMEM((K, ROW), jnp.float32),
          pltpu.VMEM((K, ROW), jnp.float32),
          pltpu.VMEM((K, ROW), jnp.float32),
          pltpu.VMEM_SHARED((NUM_NODES,), jnp.float32),
          pltpu.SemaphoreType.DMA,
      ],
  )
  parts = f(src2d, dst2d, bond2d, charge, sigma, zeros)
  return parts[0] + parts[1]


# speedup vs baseline: 258.0429x; 1.6677x over previous
import functools

import jax
import jax.numpy as jnp
from jax import lax
from jax.experimental import pallas as pl
from jax.experimental.pallas import tpu as pltpu
from jax.experimental.pallas import tpu_sc as plsc

COULOMB_K = 14.399645351950548
NUM_NODES = 100000
NUM_EDGES = 6400000

NC, NS = 2, 16
NW = NC * NS
ROW = 128
K = 8
CHUNK = K * ROW
NROWS = NUM_EDGES // ROW
NCHUNKS = NROWS // K

_INV_SQRT2 = 0.7071067811865476
_ERF_P = 0.3275911
_ERF_A1 = 0.254829592
_ERF_A2 = -0.284496736
_ERF_A3 = 1.421413741
_ERF_A4 = -1.453152027
_ERF_A5 = 1.061405429


def _edge_math(si, di, d, q, sigma_v):
  ss = plsc.load_gather(sigma_v, [si])
  sd = plsc.load_gather(sigma_v, [di])
  g2 = ss * ss + sd * sd
  ii = plsc.bitcast(g2, jnp.int32)
  ii = jnp.int32(0x5F3759DF) - lax.shift_right_logical(ii, 1)
  y = plsc.bitcast(ii, jnp.float32)
  for _ in range(2):
    y = y * (1.5 - 0.5 * g2 * y * y)
  x = d * _INV_SQRT2 * y
  u = 1.0 + _ERF_P * x
  rc = 1.0 / (u * d)
  t = d * rc
  inv_d = u * rc
  e2 = jnp.exp(-(x * x))
  p = t * (_ERF_A1 + t * (_ERF_A2 + t * (_ERF_A3 + t * (_ERF_A4 + t * _ERF_A5))))
  erf_x = 1.0 - p * e2
  pc = 1.0 + d * d * d * (-10.0 + d * (15.0 - 6.0 * d))
  return (COULOMB_K * q) * erf_x * pc * inv_d


def _body(src_hbm, dst_hbm, bond_hbm, charge_hbm, sigma_hbm, zeros_hbm,
          out_hbm, sigma_v,
          srcb0, srcb1, srcb2, dstb0, dstb1, dstb2,
          bondb0, bondb1, bondb2, qb0, qb1, qb2, potb0, potb1, potb2,
          vshared,
          lsem0, lsem1, lsem2, gsem0, gsem1, gsem2, ssem0, ssem1, ssem2):
  srcb = (srcb0, srcb1, srcb2)
  dstb = (dstb0, dstb1, dstb2)
  bondb = (bondb0, bondb1, bondb2)
  qb = (qb0, qb1, qb2)
  potb = (potb0, potb1, potb2)
  lsem = (lsem0, lsem1, lsem2)
  gsem = (gsem0, gsem1, gsem2)
  ssem = (ssem0, ssem1, ssem2)

  c = lax.axis_index("c")
  s = lax.axis_index("s")
  wid = s * NC + c

  pltpu.sync_copy(sigma_hbm, sigma_v)

  @pl.when(s == 0)
  def _():
    pltpu.sync_copy(zeros_hbm, vshared)

  plsc.subcore_barrier()

  c0 = wid * NCHUNKS // NW
  c1 = (wid + 1) * NCHUNKS // NW

  def lin_descs(i, st):
    b = i * K
    return (
        pltpu.make_async_copy(src_hbm.at[pl.ds(b, K)], srcb[st], lsem[st]),
        pltpu.make_async_copy(dst_hbm.at[pl.ds(b, K)], dstb[st], lsem[st]),
        pltpu.make_async_copy(bond_hbm.at[pl.ds(b, K)], bondb[st], lsem[st]),
    )

  def start_linear(i, st):
    for dsc in lin_descs(i, st):
      dsc.start()

  def wait_linear(i, st):
    for dsc in lin_descs(i, st):
      dsc.wait()

  def g_descs(st):
    return [
        pltpu.make_async_copy(charge_hbm.at[dstb[st].at[r]], qb[st].at[r],
                              gsem[st])
        for r in range(K)
    ]

  def start_gather(st):
    for dsc in g_descs(st):
      dsc.start()

  def wait_gather(st):
    for dsc in g_descs(st):
      dsc.wait()

  def s_descs(st):
    return [
        pltpu.make_async_copy(potb[st].at[r], vshared.at[dstb[st].at[r]],
                              ssem[st])
        for r in range(K)
    ]

  def start_scatter(st):
    for dsc in s_descs(st):
      dsc.start(add=True)

  def wait_scatter(st):
    for dsc in s_descs(st):
      dsc.wait()

  def compute(st):
    def row_body(r, carry2):
      for j in range(ROW // 16):
        sl = pl.ds(j * 16, 16)
        si = srcb[st][r, sl]
        di = dstb[st][r, sl]
        d = bondb[st][r, sl]
        q = qb[st][r, sl]
        potb[st][r, sl] = _edge_math(si, di, d, q, sigma_v)
      return carry2

    lax.fori_loop(0, K, row_body, 0)

  def chunk_step(i, st):
    @pl.when(i + 1 < c1)
    def _():
      wait_linear(i + 1, (st + 1) % 3)
      start_gather((st + 1) % 3)

    @pl.when(jnp.logical_and(i + 2 < c1, i - 1 >= c0))
    def _():
      wait_scatter((st + 2) % 3)

    @pl.when(i + 2 < c1)
    def _():
      start_linear(i + 2, (st + 2) % 3)

    wait_gather(st)
    compute(st)
    start_scatter(st)

  start_linear(c0, 0)
  start_linear(c0 + 1, 1)
  wait_linear(c0, 0)
  start_gather(0)

  n = c1 - c0
  nmacro = (n + 2) // 3

  def macro_body(m, carry):
    base = c0 + 3 * m
    for k in range(3):
      @pl.when(base + k < c1)
      def _(i=base + k, k=k):
        chunk_step(i, k)
    return carry

  lax.fori_loop(0, nmacro, macro_body, 0)

  for st in range(3):
    wait_scatter(st)

  plsc.subcore_barrier()

  @pl.when(s == 0)
  def _():
    pltpu.sync_copy(vshared, out_hbm.at[c])


@jax.jit
def kernel(charge, sigma, bond_dist, edge_index):
  src2d = edge_index[0].reshape(NROWS, ROW)
  dst2d = edge_index[1].reshape(NROWS, ROW)
  bond2d = bond_dist.reshape(NROWS, ROW)
  zeros = jnp.zeros((NUM_NODES,), jnp.float32)

  mesh = plsc.VectorSubcoreMesh(core_axis_name="c", subcore_axis_name="s")
  f = pl.kernel(
      _body,
      out_type=jax.ShapeDtypeStruct((NC, NUM_NODES), jnp.float32),
      mesh=mesh,
      compiler_params=pltpu.CompilerParams(needs_layout_passes=False),
      scratch_types=(
          [pltpu.VMEM((NUM_NODES,), jnp.float32)]
          + [pltpu.VMEM((K, ROW), jnp.int32)] * 6
          + [pltpu.VMEM((K, ROW), jnp.float32)] * 9
          + [pltpu.VMEM_SHARED((NUM_NODES,), jnp.float32)]
          + [pltpu.SemaphoreType.DMA] * 9
      ),
  )
  parts = f(src2d, dst2d, bond2d, charge, sigma, zeros)
  return parts[0] + parts[1]


# speedup vs baseline: 268.3527x; 1.0400x over previous
import functools

import jax
import jax.numpy as jnp
from jax import lax
from jax.experimental import pallas as pl
from jax.experimental.pallas import tpu as pltpu
from jax.experimental.pallas import tpu_sc as plsc

COULOMB_K = 14.399645351950548
NUM_NODES = 100000
NUM_EDGES = 6400000

NC, NS = 2, 16
NW = NC * NS
ROW = 128
K = 8
CHUNK = K * ROW
NROWS = NUM_EDGES // ROW
NCHUNKS = NROWS // K

_INV_SQRT2 = 0.7071067811865476
_E1 = 0.278393
_E2 = 0.230389
_E3 = 0.000972
_E4 = 0.078108


def _edge_math(si, di, d, q, sigma_v):
  ss = plsc.load_gather(sigma_v, [si])
  sd = plsc.load_gather(sigma_v, [di])
  g2 = ss * ss + sd * sd
  ii = plsc.bitcast(g2, jnp.int32)
  ii = jnp.int32(0x5F3759DF) - lax.shift_right_logical(ii, 1)
  y = plsc.bitcast(ii, jnp.float32)
  for _ in range(2):
    y = y * (1.5 - 0.5 * g2 * y * y)
  x = d * _INV_SQRT2 * y
  u = 1.0 + x * (_E1 + x * (_E2 + x * (_E3 + x * _E4)))
  rc = 1.0 / (u * d)
  iu = d * rc
  inv_d = u * rc
  iu2 = iu * iu
  erf_x = 1.0 - iu2 * iu2
  pc = 1.0 + d * d * d * (-10.0 + d * (15.0 - 6.0 * d))
  return (COULOMB_K * q) * erf_x * pc * inv_d


def _body(src_hbm, dst_hbm, bond_hbm, charge_hbm, sigma_hbm, zeros_hbm,
          out_hbm, sigma_v,
          srcb0, srcb1, srcb2, dstb0, dstb1, dstb2,
          bondb0, bondb1, bondb2, qb0, qb1, qb2, potb0, potb1, potb2,
          vshared,
          lsem0, lsem1, lsem2, gsem0, gsem1, gsem2, ssem0, ssem1, ssem2):
  srcb = (srcb0, srcb1, srcb2)
  dstb = (dstb0, dstb1, dstb2)
  bondb = (bondb0, bondb1, bondb2)
  qb = (qb0, qb1, qb2)
  potb = (potb0, potb1, potb2)
  lsem = (lsem0, lsem1, lsem2)
  gsem = (gsem0, gsem1, gsem2)
  ssem = (ssem0, ssem1, ssem2)

  c = lax.axis_index("c")
  s = lax.axis_index("s")
  wid = s * NC + c

  pltpu.sync_copy(sigma_hbm, sigma_v)

  @pl.when(s == 0)
  def _():
    pltpu.sync_copy(zeros_hbm, vshared)

  plsc.subcore_barrier()

  c0 = wid * NCHUNKS // NW
  c1 = (wid + 1) * NCHUNKS // NW

  def lin_descs(i, st):
    b = i * K
    return (
        pltpu.make_async_copy(src_hbm.at[pl.ds(b, K)], srcb[st], lsem[st]),
        pltpu.make_async_copy(dst_hbm.at[pl.ds(b, K)], dstb[st], lsem[st]),
        pltpu.make_async_copy(bond_hbm.at[pl.ds(b, K)], bondb[st], lsem[st]),
    )

  def start_linear(i, st):
    for dsc in lin_descs(i, st):
      dsc.start()

  def wait_linear(i, st):
    for dsc in lin_descs(i, st):
      dsc.wait()

  def g_descs(st):
    return [
        pltpu.make_async_copy(charge_hbm.at[dstb[st].at[r]], qb[st].at[r],
                              gsem[st])
        for r in range(K)
    ]

  def start_gather(st):
    for dsc in g_descs(st):
      dsc.start()

  def wait_gather(st):
    for dsc in g_descs(st):
      dsc.wait()

  def s_descs(st):
    return [
        pltpu.make_async_copy(potb[st].at[r], vshared.at[dstb[st].at[r]],
                              ssem[st])
        for r in range(K)
    ]

  def start_scatter(st):
    for dsc in s_descs(st):
      dsc.start(add=True)

  def wait_scatter(st):
    for dsc in s_descs(st):
      dsc.wait()

  def compute(st):
    def row_body(r, carry2):
      for j in range(ROW // 16):
        sl = pl.ds(j * 16, 16)
        si = srcb[st][r, sl]
        di = dstb[st][r, sl]
        d = bondb[st][r, sl]
        q = qb[st][r, sl]
        potb[st][r, sl] = _edge_math(si, di, d, q, sigma_v)
      return carry2

    lax.fori_loop(0, K, row_body, 0)

  def chunk_step(i, st):
    @pl.when(i + 1 < c1)
    def _():
      wait_linear(i + 1, (st + 1) % 3)
      start_gather((st + 1) % 3)

    @pl.when(jnp.logical_and(i + 2 < c1, i - 1 >= c0))
    def _():
      wait_scatter((st + 2) % 3)

    @pl.when(i + 2 < c1)
    def _():
      start_linear(i + 2, (st + 2) % 3)

    wait_gather(st)
    compute(st)
    start_scatter(st)

  start_linear(c0, 0)
  start_linear(c0 + 1, 1)
  wait_linear(c0, 0)
  start_gather(0)

  n = c1 - c0
  nmacro = (n + 2) // 3

  def macro_body(m, carry):
    base = c0 + 3 * m
    for k in range(3):
      @pl.when(base + k < c1)
      def _(i=base + k, k=k):
        chunk_step(i, k)
    return carry

  lax.fori_loop(0, nmacro, macro_body, 0)

  for st in range(3):
    wait_scatter(st)

  plsc.subcore_barrier()

  @pl.when(s == 0)
  def _():
    pltpu.sync_copy(vshared, out_hbm.at[c])


@jax.jit
def kernel(charge, sigma, bond_dist, edge_index):
  src2d = edge_index[0].reshape(NROWS, ROW)
  dst2d = edge_index[1].reshape(NROWS, ROW)
  bond2d = bond_dist.reshape(NROWS, ROW)
  zeros = jnp.zeros((NUM_NODES,), jnp.float32)

  mesh = plsc.VectorSubcoreMesh(core_axis_name="c", subcore_axis_name="s")
  f = pl.kernel(
      _body,
      out_type=jax.ShapeDtypeStruct((NC, NUM_NODES), jnp.float32),
      mesh=mesh,
      compiler_params=pltpu.CompilerParams(needs_layout_passes=False),
      scratch_types=(
          [pltpu.VMEM((NUM_NODES,), jnp.float32)]
          + [pltpu.VMEM((K, ROW), jnp.int32)] * 6
          + [pltpu.VMEM((K, ROW), jnp.float32)] * 9
          + [pltpu.VMEM_SHARED((NUM_NODES,), jnp.float32)]
          + [pltpu.SemaphoreType.DMA] * 9
      ),
  )
  parts = f(src2d, dst2d, bond2d, charge, sigma, zeros)
  return parts[0] + parts[1]


# speedup vs baseline: 271.3534x; 1.0112x over previous
import jax
import jax.numpy as jnp
from jax import lax
from jax.experimental import pallas as pl
from jax.experimental.pallas import tpu as pltpu
from jax.experimental.pallas import tpu_sc as plsc

COULOMB_K = 14.399645351950548
NUM_NODES = 100000
NUM_EDGES = 6400000

NC, NS = 2, 16
NW = NC * NS
CHUNK = 1024
NCHUNKS = NUM_EDGES // CHUNK

_INV_SQRT2 = 0.7071067811865476
_E1 = 0.278393
_E2 = 0.230389
_E3 = 0.000972
_E4 = 0.078108


def _edge_math(si, di, d, q, sigma_v):
  ss = plsc.load_gather(sigma_v, [si])
  sd = plsc.load_gather(sigma_v, [di])
  g2 = ss * ss + sd * sd
  ii = plsc.bitcast(g2, jnp.int32)
  ii = jnp.int32(0x5F3759DF) - lax.shift_right_logical(ii, 1)
  y = plsc.bitcast(ii, jnp.float32)
  for _ in range(2):
    y = y * (1.5 - 0.5 * g2 * y * y)
  x = d * _INV_SQRT2 * y
  u = 1.0 + x * (_E1 + x * (_E2 + x * (_E3 + x * _E4)))
  rc = 1.0 / (u * d)
  iu = d * rc
  inv_d = u * rc
  iu2 = iu * iu
  erf_x = 1.0 - iu2 * iu2
  pc = 1.0 + d * d * d * (-10.0 + d * (15.0 - 6.0 * d))
  return (COULOMB_K * q) * erf_x * pc * inv_d


def _body(src_hbm, dst_hbm, bond_hbm, charge_hbm, sigma_hbm, zeros_hbm,
          out_hbm, sigma_v,
          srcb0, srcb1, srcb2, dstb0, dstb1, dstb2,
          bondb0, bondb1, bondb2, qb0, qb1, qb2, potb0, potb1, potb2,
          vshared,
          lsem0, lsem1, lsem2, gsem0, gsem1, gsem2, ssem0, ssem1, ssem2):
  srcb = (srcb0, srcb1, srcb2)
  dstb = (dstb0, dstb1, dstb2)
  bondb = (bondb0, bondb1, bondb2)
  qb = (qb0, qb1, qb2)
  potb = (potb0, potb1, potb2)
  lsem = (lsem0, lsem1, lsem2)
  gsem = (gsem0, gsem1, gsem2)
  ssem = (ssem0, ssem1, ssem2)

  c = lax.axis_index("c")
  s = lax.axis_index("s")
  wid = s * NC + c

  pltpu.sync_copy(sigma_hbm, sigma_v)

  @pl.when(s == 0)
  def _():
    pltpu.sync_copy(zeros_hbm, vshared)

  plsc.subcore_barrier()

  c0 = wid * NCHUNKS // NW
  c1 = (wid + 1) * NCHUNKS // NW

  def lin_descs(i, st):
    b = i * CHUNK
    return (
        pltpu.make_async_copy(src_hbm.at[pl.ds(b, CHUNK)], srcb[st], lsem[st]),
        pltpu.make_async_copy(dst_hbm.at[pl.ds(b, CHUNK)], dstb[st], lsem[st]),
        pltpu.make_async_copy(bond_hbm.at[pl.ds(b, CHUNK)], bondb[st],
                              lsem[st]),
    )

  def start_linear(i, st):
    for dsc in lin_descs(i, st):
      dsc.start()

  def wait_linear(i, st):
    for dsc in lin_descs(i, st):
      dsc.wait()

  def g_desc(st):
    return pltpu.make_async_copy(charge_hbm.at[dstb[st]], qb[st], gsem[st])

  def s_desc(st):
    return pltpu.make_async_copy(potb[st], vshared.at[dstb[st]], ssem[st])

  def compute(st):
    def vec_body(v, carry2):
      sl = pl.ds(v * 16, 16)
      si = srcb[st][sl]
      di = dstb[st][sl]
      d = bondb[st][sl]
      q = qb[st][sl]
      potb[st][sl] = _edge_math(si, di, d, q, sigma_v)
      return carry2

    lax.fori_loop(0, CHUNK // 16, vec_body, 0)

  def chunk_step(i, st):
    @pl.when(i + 1 < c1)
    def _():
      wait_linear(i + 1, (st + 1) % 3)
      g_desc((st + 1) % 3).start()

    @pl.when(jnp.logical_and(i + 2 < c1, i - 1 >= c0))
    def _():
      s_desc((st + 2) % 3).wait()

    @pl.when(i + 2 < c1)
    def _():
      start_linear(i + 2, (st + 2) % 3)

    g_desc(st).wait()
    compute(st)
    s_desc(st).start(add=True)

  start_linear(c0, 0)
  start_linear(c0 + 1, 1)
  wait_linear(c0, 0)
  g_desc(0).start()

  n = c1 - c0
  nmacro = (n + 2) // 3

  def macro_body(m, carry):
    base = c0 + 3 * m
    for k in range(3):
      @pl.when(base + k < c1)
      def _(i=base + k, k=k):
        chunk_step(i, k)
    return carry

  lax.fori_loop(0, nmacro, macro_body, 0)

  for st in range(3):
    s_desc(st).wait()

  plsc.subcore_barrier()

  @pl.when(s == 0)
  def _():
    pltpu.sync_copy(vshared, out_hbm.at[c])


@jax.jit
def kernel(charge, sigma, bond_dist, edge_index):
  src = edge_index[0]
  dst = edge_index[1]
  zeros = jnp.zeros((NUM_NODES,), jnp.float32)

  mesh = plsc.VectorSubcoreMesh(core_axis_name="c", subcore_axis_name="s")
  f = pl.kernel(
      _body,
      out_type=jax.ShapeDtypeStruct((NC, NUM_NODES), jnp.float32),
      mesh=mesh,
      compiler_params=pltpu.CompilerParams(needs_layout_passes=False),
      scratch_types=(
          [pltpu.VMEM((NUM_NODES,), jnp.float32)]
          + [pltpu.VMEM((CHUNK,), jnp.int32)] * 6
          + [pltpu.VMEM((CHUNK,), jnp.float32)] * 9
          + [pltpu.VMEM_SHARED((NUM_NODES,), jnp.float32)]
          + [pltpu.SemaphoreType.DMA] * 9
      ),
  )
  parts = f(src, dst, bond_dist, charge, sigma, zeros)
  return parts[0] + parts[1]


# speedup vs baseline: 474.6082x; 1.7490x over previous
import jax
import jax.numpy as jnp
from jax import lax
from jax.experimental import pallas as pl
from jax.experimental.pallas import tpu as pltpu
from jax.experimental.pallas import tpu_sc as plsc

COULOMB_K = 14.399645351950548
NUM_NODES = 100000
NUM_EDGES = 6400000

NC, NS = 2, 16
NW = NC * NS
CHUNK = 1024
NCHUNKS = NUM_EDGES // CHUNK

_INV_SQRT2 = 0.7071067811865476
_E1 = 0.278393
_E2 = 0.230389
_E3 = 0.000972
_E4 = 0.078108


def _edge_math(si, di, d, q, sigma_v):
  ss = plsc.load_gather(sigma_v, [si])
  sd = plsc.load_gather(sigma_v, [di])
  g2 = ss * ss + sd * sd
  ii = plsc.bitcast(g2, jnp.int32)
  ii = jnp.int32(0x5F3759DF) - lax.shift_right_logical(ii, 1)
  y = plsc.bitcast(ii, jnp.float32)
  for _ in range(2):
    y = y * (1.5 - 0.5 * g2 * y * y)
  x = d * _INV_SQRT2 * y
  u = 1.0 + x * (_E1 + x * (_E2 + x * (_E3 + x * _E4)))
  rc = 1.0 / (u * d)
  iu = d * rc
  inv_d = u * rc
  iu2 = iu * iu
  erf_x = 1.0 - iu2 * iu2
  pc = 1.0 + d * d * d * (-10.0 + d * (15.0 - 6.0 * d))
  return (COULOMB_K * q) * erf_x * pc * inv_d


def _body(src_hbm, dst_hbm, bond_hbm, charge_hbm, sigma_hbm, zeros_hbm,
          out_hbm, sigma_v,
          srcb0, srcb1, srcb2, dstb0, dstb1, dstb2,
          bondb0, bondb1, bondb2, qb0, qb1, qb2, potb0, potb1, potb2,
          vshared,
          lsem0, lsem1, lsem2, gsem0, gsem1, gsem2, ssem0, ssem1, ssem2):
  srcb = (srcb0, srcb1, srcb2)
  dstb = (dstb0, dstb1, dstb2)
  bondb = (bondb0, bondb1, bondb2)
  qb = (qb0, qb1, qb2)
  potb = (potb0, potb1, potb2)
  lsem = (lsem0, lsem1, lsem2)
  gsem = (gsem0, gsem1, gsem2)
  ssem = (ssem0, ssem1, ssem2)

  c = lax.axis_index("c")
  s = lax.axis_index("s")
  wid = s * NC + c

  pltpu.sync_copy(sigma_hbm, sigma_v)

  @pl.when(s == 0)
  def _():
    pltpu.sync_copy(zeros_hbm, vshared)

  plsc.subcore_barrier()

  c0 = wid * NCHUNKS // NW
  c1 = (wid + 1) * NCHUNKS // NW

  def lin_descs(i, st):
    b = i * CHUNK
    return (
        pltpu.make_async_copy(src_hbm.at[pl.ds(b, CHUNK)], srcb[st], lsem[st]),
        pltpu.make_async_copy(dst_hbm.at[pl.ds(b, CHUNK)], dstb[st], lsem[st]),
        pltpu.make_async_copy(bond_hbm.at[pl.ds(b, CHUNK)], bondb[st],
                              lsem[st]),
    )

  def start_linear(i, st):
    for dsc in lin_descs(i, st):
      dsc.start()

  def wait_linear(i, st):
    for dsc in lin_descs(i, st):
      dsc.wait()

  def g_desc(st):
    return pltpu.make_async_copy(charge_hbm.at[dstb[st]], qb[st], gsem[st])

  def s_desc(st):
    return pltpu.make_async_copy(potb[st], vshared.at[dstb[st]], ssem[st])

  def compute(st):
    @plsc.parallel_loop(0, CHUNK, step=16, unroll=4)
    def _(off):
      sl = pl.ds(off, 16)
      si = srcb[st][sl]
      di = dstb[st][sl]
      d = bondb[st][sl]
      q = qb[st][sl]
      potb[st][sl] = _edge_math(si, di, d, q, sigma_v)

  def chunk_step(i, st):
    @pl.when(i + 1 < c1)
    def _():
      wait_linear(i + 1, (st + 1) % 3)
      g_desc((st + 1) % 3).start()

    @pl.when(jnp.logical_and(i + 2 < c1, i - 1 >= c0))
    def _():
      s_desc((st + 2) % 3).wait()

    @pl.when(i + 2 < c1)
    def _():
      start_linear(i + 2, (st + 2) % 3)

    g_desc(st).wait()
    compute(st)
    s_desc(st).start(add=True)

  start_linear(c0, 0)
  start_linear(c0 + 1, 1)
  wait_linear(c0, 0)
  g_desc(0).start()

  n = c1 - c0
  nmacro = (n + 2) // 3

  def macro_body(m, carry):
    base = c0 + 3 * m
    for k in range(3):
      @pl.when(base + k < c1)
      def _(i=base + k, k=k):
        chunk_step(i, k)
    return carry

  lax.fori_loop(0, nmacro, macro_body, 0)

  for st in range(3):
    s_desc(st).wait()

  plsc.subcore_barrier()

  @pl.when(s == 0)
  def _():
    pltpu.sync_copy(vshared, out_hbm.at[c])


@jax.jit
def kernel(charge, sigma, bond_dist, edge_index):
  src = edge_index[0]
  dst = edge_index[1]
  zeros = jnp.zeros((NUM_NODES,), jnp.float32)

  mesh = plsc.VectorSubcoreMesh(core_axis_name="c", subcore_axis_name="s")
  f = pl.kernel(
      _body,
      out_type=jax.ShapeDtypeStruct((NC, NUM_NODES), jnp.float32),
      mesh=mesh,
      compiler_params=pltpu.CompilerParams(needs_layout_passes=False),
      scratch_types=(
          [pltpu.VMEM((NUM_NODES,), jnp.float32)]
          + [pltpu.VMEM((CHUNK,), jnp.int32)] * 6
          + [pltpu.VMEM((CHUNK,), jnp.float32)] * 9
          + [pltpu.VMEM_SHARED((NUM_NODES,), jnp.float32)]
          + [pltpu.SemaphoreType.DMA] * 9
      ),
  )
  parts = f(src, dst, bond_dist, charge, sigma, zeros)
  return parts[0] + parts[1]


# speedup vs baseline: 476.5727x; 1.0041x over previous
import jax
import jax.numpy as jnp
from jax import lax
from jax.experimental import pallas as pl
from jax.experimental.pallas import tpu as pltpu
from jax.experimental.pallas import tpu_sc as plsc

COULOMB_K = 14.399645351950548
NUM_NODES = 100000
NUM_EDGES = 6400000

NC, NS = 2, 16
NW = NC * NS
CHUNK = 1024
NCHUNKS = NUM_EDGES // CHUNK

_INV_SQRT2 = 0.7071067811865476
_E1 = 0.278393
_E2 = 0.230389
_E3 = 0.000972
_E4 = 0.078108


def _edge_math(si, di, d, q, sigma_v):
  ss = plsc.load_gather(sigma_v, [si])
  sd = plsc.load_gather(sigma_v, [di])
  g2 = ss * ss + sd * sd
  ii = plsc.bitcast(g2, jnp.int32)
  ii = jnp.int32(0x5F3759DF) - lax.shift_right_logical(ii, 1)
  y = plsc.bitcast(ii, jnp.float32)
  for _ in range(2):
    y = y * (1.5 - 0.5 * g2 * y * y)
  x = d * _INV_SQRT2 * y
  u = 1.0 + x * (_E1 + x * (_E2 + x * (_E3 + x * _E4)))
  rc = 1.0 / (u * d)
  iu = d * rc
  inv_d = u * rc
  iu2 = iu * iu
  erf_x = 1.0 - iu2 * iu2
  pc = 1.0 + d * d * d * (-10.0 + d * (15.0 - 6.0 * d))
  return (COULOMB_K * q) * erf_x * pc * inv_d


def _body(src_hbm, dst_hbm, bond_hbm, charge_hbm, sigma_hbm, zeros_hbm,
          out_hbm, sigma_v,
          srcb0, srcb1, srcb2, dstb0, dstb1, dstb2,
          bondb0, bondb1, bondb2, qb0, qb1, qb2, potb0, potb1, potb2,
          vshared,
          lsem0, lsem1, lsem2, gsem0, gsem1, gsem2, ssem0, ssem1, ssem2):
  srcb = (srcb0, srcb1, srcb2)
  dstb = (dstb0, dstb1, dstb2)
  bondb = (bondb0, bondb1, bondb2)
  qb = (qb0, qb1, qb2)
  potb = (potb0, potb1, potb2)
  lsem = (lsem0, lsem1, lsem2)
  gsem = (gsem0, gsem1, gsem2)
  ssem = (ssem0, ssem1, ssem2)

  c = lax.axis_index("c")
  s = lax.axis_index("s")
  wid = s * NC + c

  pltpu.sync_copy(sigma_hbm, sigma_v)

  @pl.when(s == 0)
  def _():
    pltpu.sync_copy(zeros_hbm, vshared)

  plsc.subcore_barrier()

  c0 = wid * NCHUNKS // NW
  c1 = (wid + 1) * NCHUNKS // NW

  def lin_descs(i, st):
    b = i * CHUNK
    return (
        pltpu.make_async_copy(src_hbm.at[pl.ds(b, CHUNK)], srcb[st], lsem[st]),
        pltpu.make_async_copy(dst_hbm.at[pl.ds(b, CHUNK)], dstb[st], lsem[st]),
        pltpu.make_async_copy(bond_hbm.at[pl.ds(b, CHUNK)], bondb[st],
                              lsem[st]),
    )

  def start_linear(i, st):
    for dsc in lin_descs(i, st):
      dsc.start()

  def wait_linear(i, st):
    for dsc in lin_descs(i, st):
      dsc.wait()

  def g_desc(st):
    return pltpu.make_async_copy(charge_hbm.at[dstb[st]], qb[st], gsem[st])

  def s_desc(st):
    return pltpu.make_async_copy(potb[st], vshared.at[dstb[st]], ssem[st])

  def compute(st):
    @plsc.parallel_loop(0, CHUNK, step=16, unroll=8)
    def _(off):
      sl = pl.ds(off, 16)
      si = srcb[st][sl]
      di = dstb[st][sl]
      d = bondb[st][sl]
      q = qb[st][sl]
      potb[st][sl] = _edge_math(si, di, d, q, sigma_v)

  def chunk_step(i, st):
    @pl.when(i + 1 < c1)
    def _():
      wait_linear(i + 1, (st + 1) % 3)
      g_desc((st + 1) % 3).start()

    @pl.when(jnp.logical_and(i + 2 < c1, i - 1 >= c0))
    def _():
      s_desc((st + 2) % 3).wait()

    @pl.when(i + 2 < c1)
    def _():
      start_linear(i + 2, (st + 2) % 3)

    g_desc(st).wait()
    compute(st)
    s_desc(st).start(add=True)

  start_linear(c0, 0)
  start_linear(c0 + 1, 1)
  wait_linear(c0, 0)
  g_desc(0).start()

  n = c1 - c0
  nmacro = (n + 2) // 3

  def macro_body(m, carry):
    base = c0 + 3 * m
    for k in range(3):
      @pl.when(base + k < c1)
      def _(i=base + k, k=k):
        chunk_step(i, k)
    return carry

  lax.fori_loop(0, nmacro, macro_body, 0)

  for st in range(3):
    s_desc(st).wait()

  plsc.subcore_barrier()

  @pl.when(s == 0)
  def _():
    pltpu.sync_copy(vshared, out_hbm.at[c])


@jax.jit
def kernel(charge, sigma, bond_dist, edge_index):
  src = edge_index[0]
  dst = edge_index[1]
  zeros = jnp.zeros((NUM_NODES,), jnp.float32)

  mesh = plsc.VectorSubcoreMesh(core_axis_name="c", subcore_axis_name="s")
  f = pl.kernel(
      _body,
      out_type=jax.ShapeDtypeStruct((NC, NUM_NODES), jnp.float32),
      mesh=mesh,
      compiler_params=pltpu.CompilerParams(needs_layout_passes=False),
      scratch_types=(
          [pltpu.VMEM((NUM_NODES,), jnp.float32)]
          + [pltpu.VMEM((CHUNK,), jnp.int32)] * 6
          + [pltpu.VMEM((CHUNK,), jnp.float32)] * 9
          + [pltpu.VMEM_SHARED((NUM_NODES,), jnp.float32)]
          + [pltpu.SemaphoreType.DMA] * 9
      ),
  )
  parts = f(src, dst, bond_dist, charge, sigma, zeros)
  return parts[0] + parts[1]


# speedup vs baseline: 677.1411x; 1.4209x over previous
import jax
import jax.numpy as jnp
from jax import lax
from jax.experimental import pallas as pl
from jax.experimental.pallas import tpu as pltpu
from jax.experimental.pallas import tpu_sc as plsc

COULOMB_K = 14.399645351950548
NUM_NODES = 100000
NUM_EDGES = 6400000

NC, NS = 2, 16
NW = NC * NS
CHUNK = 2048
NCHUNKS = NUM_EDGES // CHUNK

NPAD = 49 * CHUNK

_INV_SQRT2 = 0.7071067811865476
_E1 = 0.278393
_E2 = 0.230389
_E3 = 0.000972
_E4 = 0.078108


def _edge_math(si, di, d, sigma_v):
  ss = plsc.load_gather(sigma_v, [si])
  sd = plsc.load_gather(sigma_v, [di])
  g2 = ss * ss + sd * sd
  ii = plsc.bitcast(g2, jnp.int32)
  ii = jnp.int32(0x5F3759DF) - lax.shift_right_logical(ii, 1)
  y = plsc.bitcast(ii, jnp.float32)
  for _ in range(2):
    y = y * (1.5 - 0.5 * g2 * y * y)
  x = d * _INV_SQRT2 * y
  u = 1.0 + x * (_E1 + x * (_E2 + x * (_E3 + x * _E4)))
  rc = 1.0 / (u * d)
  iu = d * rc
  inv_d = u * rc
  iu2 = iu * iu
  erf_x = 1.0 - iu2 * iu2
  pc = 1.0 + d * d * d * (-10.0 + d * (15.0 - 6.0 * d))
  return (COULOMB_K * erf_x) * pc * inv_d


def _body(src_hbm, dst_hbm, bond_hbm, charge_hbm, sigma_hbm, zeros_hbm,
          out_hbm, sigma_v,
          srcb0, srcb1, srcb2, dstb0, dstb1, dstb2,
          bondb0, bondb1, bondb2, potb0, potb1, potb2,
          vshared,
          lsem0, lsem1, lsem2, ssem0, ssem1, ssem2):
  srcb = (srcb0, srcb1, srcb2)
  dstb = (dstb0, dstb1, dstb2)
  bondb = (bondb0, bondb1, bondb2)
  potb = (potb0, potb1, potb2)
  lsem = (lsem0, lsem1, lsem2)
  ssem = (ssem0, ssem1, ssem2)

  c = lax.axis_index("c")
  s = lax.axis_index("s")
  wid = s * NC + c

  pltpu.sync_copy(sigma_hbm, sigma_v)

  @pl.when(s == 0)
  def _():
    pltpu.sync_copy(zeros_hbm, vshared)

  plsc.subcore_barrier()

  c0 = wid * NCHUNKS // NW
  c1 = (wid + 1) * NCHUNKS // NW

  def lin_descs(i, st):
    b = i * CHUNK
    return (
        pltpu.make_async_copy(src_hbm.at[pl.ds(b, CHUNK)], srcb[st], lsem[st]),
        pltpu.make_async_copy(dst_hbm.at[pl.ds(b, CHUNK)], dstb[st], lsem[st]),
        pltpu.make_async_copy(bond_hbm.at[pl.ds(b, CHUNK)], bondb[st],
                              lsem[st]),
    )

  def start_linear(i, st):
    for dsc in lin_descs(i, st):
      dsc.start()

  def wait_linear(i, st):
    for dsc in lin_descs(i, st):
      dsc.wait()

  def s_desc(st):
    return pltpu.make_async_copy(potb[st], vshared.at[dstb[st]], ssem[st])

  def compute(st):
    @plsc.parallel_loop(0, CHUNK, step=16, unroll=8)
    def _(off):
      sl = pl.ds(off, 16)
      si = srcb[st][sl]
      di = dstb[st][sl]
      d = bondb[st][sl]
      potb[st][sl] = _edge_math(si, di, d, sigma_v)

  def chunk_step(i, st):
    @pl.when(jnp.logical_and(i + 2 < c1, i - 1 >= c0))
    def _():
      s_desc((st + 2) % 3).wait()

    @pl.when(i + 2 < c1)
    def _():
      start_linear(i + 2, (st + 2) % 3)

    wait_linear(i, st)
    compute(st)
    s_desc(st).start(add=True)

  start_linear(c0, 0)
  start_linear(c0 + 1, 1)

  n = c1 - c0
  nmacro = (n + 2) // 3

  def macro_body(m, carry):
    base = c0 + 3 * m
    for k in range(3):
      @pl.when(base + k < c1)
      def _(i=base + k, k=k):
        chunk_step(i, k)
    return carry

  lax.fori_loop(0, nmacro, macro_body, 0)

  for st in range(3):
    s_desc(st).wait()

  plsc.subcore_barrier()

  out_row = out_hbm.at[c]

  def scale_slice(base):
    pltpu.sync_copy(vshared.at[pl.ds(base, CHUNK)], potb0)
    pltpu.sync_copy(charge_hbm.at[pl.ds(base, CHUNK)], bondb0)

    @plsc.parallel_loop(0, CHUNK, step=16, unroll=4)
    def _(off):
      sl = pl.ds(off, 16)
      potb0[sl] = potb0[sl] * bondb0[sl]

    pltpu.sync_copy(potb0, out_row.at[pl.ds(base, CHUNK)])

  for t in range(3):
    scale_slice((s * 3 + t) * CHUNK)

  @pl.when(s == 0)
  def _():
    scale_slice(48 * CHUNK)


@jax.jit
def kernel(charge, sigma, bond_dist, edge_index):
  src = edge_index[0]
  dst = edge_index[1]
  charge_p = jnp.pad(charge, (0, NPAD - NUM_NODES))
  zeros = jnp.zeros((NPAD,), jnp.float32)

  mesh = plsc.VectorSubcoreMesh(core_axis_name="c", subcore_axis_name="s")
  f = pl.kernel(
      _body,
      out_type=jax.ShapeDtypeStruct((NC, NPAD), jnp.float32),
      mesh=mesh,
      compiler_params=pltpu.CompilerParams(needs_layout_passes=False),
      scratch_types=(
          [pltpu.VMEM((NUM_NODES,), jnp.float32)]
          + [pltpu.VMEM((CHUNK,), jnp.int32)] * 6
          + [pltpu.VMEM((CHUNK,), jnp.float32)] * 6
          + [pltpu.VMEM_SHARED((NPAD,), jnp.float32)]
          + [pltpu.SemaphoreType.DMA] * 6
      ),
  )
  parts = f(src, dst, bond_dist, charge_p, sigma, zeros)
  return (parts[0] + parts[1])[:NUM_NODES]


# speedup vs baseline: 679.6100x; 1.0036x over previous
import jax
import jax.numpy as jnp
from jax import lax
from jax.experimental import pallas as pl
from jax.experimental.pallas import tpu as pltpu
from jax.experimental.pallas import tpu_sc as plsc

COULOMB_K = 14.399645351950548
NUM_NODES = 100000
NUM_EDGES = 6400000

NC, NS = 2, 16
NW = NC * NS
CHUNK = 2048
NCHUNKS = NUM_EDGES // CHUNK

NPAD = 49 * CHUNK

_INV_SQRT2 = 0.7071067811865476
_E1 = 0.278393
_E2 = 0.230389
_E3 = 0.000972
_E4 = 0.078108


def _edge_math(si, di, d, sigma_v):
  ss = plsc.load_gather(sigma_v, [si])
  sd = plsc.load_gather(sigma_v, [di])
  g2 = ss * ss + sd * sd
  ii = plsc.bitcast(g2, jnp.int32)
  ii = jnp.int32(0x5F3759DF) - lax.shift_right_logical(ii, 1)
  y = plsc.bitcast(ii, jnp.float32)
  for _ in range(2):
    y = y * (1.5 - 0.5 * g2 * y * y)
  x = d * _INV_SQRT2 * y
  u = 1.0 + x * (_E1 + x * (_E2 + x * (_E3 + x * _E4)))
  rc = 1.0 / (u * d)
  iu = d * rc
  inv_d = u * rc
  iu2 = iu * iu
  erf_x = 1.0 - iu2 * iu2
  pc = 1.0 + d * d * d * (-10.0 + d * (15.0 - 6.0 * d))
  return (COULOMB_K * erf_x) * pc * inv_d


def _body(src_hbm, dst_hbm, bond_hbm, charge_hbm, sigma_hbm, zeros_hbm,
          out_hbm, sigma_v,
          srcb0, srcb1, srcb2, dstb0, dstb1, dstb2,
          bondb0, bondb1, bondb2, potb0, potb1, potb2,
          vshared,
          lsem0, lsem1, lsem2, ssem0, ssem1, ssem2):
  srcb = (srcb0, srcb1, srcb2)
  dstb = (dstb0, dstb1, dstb2)
  bondb = (bondb0, bondb1, bondb2)
  potb = (potb0, potb1, potb2)
  lsem = (lsem0, lsem1, lsem2)
  ssem = (ssem0, ssem1, ssem2)

  c = lax.axis_index("c")
  s = lax.axis_index("s")
  wid = s * NC + c

  c0 = wid * NCHUNKS // NW
  c1 = (wid + 1) * NCHUNKS // NW

  def lin_descs(i, st):
    b = i * CHUNK
    return (
        pltpu.make_async_copy(src_hbm.at[pl.ds(b, CHUNK)], srcb[st], lsem[st]),
        pltpu.make_async_copy(dst_hbm.at[pl.ds(b, CHUNK)], dstb[st], lsem[st]),
        pltpu.make_async_copy(bond_hbm.at[pl.ds(b, CHUNK)], bondb[st],
                              lsem[st]),
    )

  def start_linear(i, st):
    for dsc in lin_descs(i, st):
      dsc.start()

  def wait_linear(i, st):
    for dsc in lin_descs(i, st):
      dsc.wait()

  def s_desc(st):
    return pltpu.make_async_copy(potb[st], vshared.at[dstb[st]], ssem[st])

  def compute(st):
    @plsc.parallel_loop(0, CHUNK, step=16, unroll=8)
    def _(off):
      sl = pl.ds(off, 16)
      si = srcb[st][sl]
      di = dstb[st][sl]
      d = bondb[st][sl]
      potb[st][sl] = _edge_math(si, di, d, sigma_v)

  def chunk_step(i, st):
    @pl.when(jnp.logical_and(i + 2 < c1, i - 1 >= c0))
    def _():
      s_desc((st + 2) % 3).wait()

    @pl.when(i + 2 < c1)
    def _():
      start_linear(i + 2, (st + 2) % 3)

    wait_linear(i, st)
    compute(st)
    s_desc(st).start(add=True)

  start_linear(c0, 0)
  start_linear(c0 + 1, 1)

  pltpu.sync_copy(sigma_hbm, sigma_v)

  @pl.when(s == 0)
  def _():
    pltpu.sync_copy(zeros_hbm, vshared)

  plsc.subcore_barrier()

  n = c1 - c0
  nmacro = (n + 2) // 3

  def macro_body(m, carry):
    base = c0 + 3 * m
    for k in range(3):
      @pl.when(base + k < c1)
      def _(i=base + k, k=k):
        chunk_step(i, k)
    return carry

  lax.fori_loop(0, nmacro, macro_body, 0)

  for st in range(3):
    s_desc(st).wait()

  plsc.subcore_barrier()

  out_row = out_hbm.at[c]

  def scale_slice(base):
    pltpu.sync_copy(vshared.at[pl.ds(base, CHUNK)], potb0)
    pltpu.sync_copy(charge_hbm.at[pl.ds(base, CHUNK)], bondb0)

    @plsc.parallel_loop(0, CHUNK, step=16, unroll=4)
    def _(off):
      sl = pl.ds(off, 16)
      potb0[sl] = potb0[sl] * bondb0[sl]

    pltpu.sync_copy(potb0, out_row.at[pl.ds(base, CHUNK)])

  for t in range(3):
    scale_slice((s * 3 + t) * CHUNK)

  @pl.when(s == 0)
  def _():
    scale_slice(48 * CHUNK)


@jax.jit
def kernel(charge, sigma, bond_dist, edge_index):
  src = edge_index[0]
  dst = edge_index[1]
  charge_p = jnp.pad(charge, (0, NPAD - NUM_NODES))
  zeros = jnp.zeros((NPAD,), jnp.float32)

  mesh = plsc.VectorSubcoreMesh(core_axis_name="c", subcore_axis_name="s")
  f = pl.kernel(
      _body,
      out_type=jax.ShapeDtypeStruct((NC, NPAD), jnp.float32),
      mesh=mesh,
      compiler_params=pltpu.CompilerParams(needs_layout_passes=False),
      scratch_types=(
          [pltpu.VMEM((NUM_NODES,), jnp.float32)]
          + [pltpu.VMEM((CHUNK,), jnp.int32)] * 6
          + [pltpu.VMEM((CHUNK,), jnp.float32)] * 6
          + [pltpu.VMEM_SHARED((NPAD,), jnp.float32)]
          + [pltpu.SemaphoreType.DMA] * 6
      ),
  )
  parts = f(src, dst, bond_dist, charge_p, sigma, zeros)
  return (parts[0] + parts[1])[:NUM_NODES]
